# baseline (speedup 1.0000x reference)
import jax
import jax.numpy as jnp
from jax import lax
from jax.experimental import pallas as pl
from jax.experimental.pallas import tpu as pltpu
from jax.experimental.pallas import tpu_sc as plsc

N = 10000
E = 320000
D = 128
NC = 2
NS = 16
NW = NC * NS
CHUNK = 80
NP = 10240
GRP = 8
CH = 128
NG = CH // GRP
EPW = CH * CHUNK
ROWS_PER_TILE = NP // NS


def _mesh():
    return plsc.VectorSubcoreMesh(core_axis_name="c", subcore_axis_name="s")


def _count_body(dst_hbm, out_hbm, idx_v, acc_v):
    c = lax.axis_index("c")
    s = lax.axis_index("s")
    wid = c * NS + s
    pltpu.sync_copy(dst_hbm.at[wid], idx_v)

    def zero(i, carry):
        acc_v[pl.ds(pl.multiple_of(i * 16, 16), 16)] = jnp.zeros((16,), jnp.float32)
        return carry

    lax.fori_loop(0, NP // 16, zero, 0)
    ones = jnp.ones((16,), jnp.float32)

    def body(i, carry):
        idx = idx_v[pl.ds(pl.multiple_of(i * 16, 16), 16)]
        plsc.addupdate_scatter(acc_v, [idx], ones)
        return carry

    lax.fori_loop(0, EPW // 16, body, 0)
    pltpu.sync_copy(acc_v, out_hbm.at[wid])


def _sc_count(dst2):
    return pl.kernel(
        _count_body,
        out_type=jax.ShapeDtypeStruct((NW, NP), jnp.float32),
        mesh=_mesh(),
        scratch_types=[
            pltpu.VMEM((EPW,), jnp.int32),
            pltpu.VMEM((NP,), jnp.float32),
        ],
        compiler_params=pltpu.CompilerParams(needs_layout_passes=False),
    )(dst2)


def _scatter_body(g_hbm, src_hbm, dst_hbm, zeros_hbm, out_hbm,
                  isrc_v, idst_v, rows_v, accum_sh, gsems, ssem, isem):
    c = lax.axis_index("c")
    s = lax.axis_index("s")
    wid = c * NS + s
    sl = pl.ds(s * ROWS_PER_TILE, ROWS_PER_TILE)

    def idx_group(grp, pbuf):
        return (
            pltpu.make_async_copy(
                src_hbm.at[wid, pl.ds(grp * GRP, GRP)], isrc_v.at[pbuf], isem),
            pltpu.make_async_copy(
                dst_hbm.at[wid, pl.ds(grp * GRP, GRP)], idst_v.at[pbuf], isem),
        )

    def gather_desc(pg, k, buf, par):
        return pltpu.make_async_copy(
            g_hbm.at[isrc_v.at[pg, k]], rows_v.at[buf], gsems.at[par])

    def scatter_desc(pg, k, buf):
        return pltpu.make_async_copy(
            rows_v.at[buf], accum_sh.at[idst_v.at[pg, k]], ssem)

    for d_ in idx_group(0, 0):
        d_.start()
    pltpu.sync_copy(zeros_hbm.at[sl], accum_sh.at[sl])
    for d_ in idx_group(0, 0):
        d_.wait()
    for d_ in idx_group(1, 1):
        d_.start()
    gather_desc(0, 0, 0, 0).start()
    gather_desc(0, 1, 1, 1).start()
    gather_desc(0, 2, 2, 2).start()
    plsc.subcore_barrier()

    def body(j, carry):
        grp = lax.div(j, GRP)
        k = lax.rem(j, GRP)
        pg = lax.rem(grp, 2)
        buf = lax.rem(j, 4)
        par = lax.rem(j, 3)

        @pl.when(jnp.logical_and(jnp.logical_and(k == 0, j > 0), grp + 1 < NG))
        def _():
            for d_ in idx_group(grp + 1, lax.rem(grp + 1, 2)):
                d_.start()

        @pl.when(j > 0)
        def _():
            jm = j - 1
            scatter_desc(lax.rem(lax.div(jm, GRP), 2), lax.rem(jm, GRP),
                         lax.rem(jm, 4)).wait()

        gather_desc(pg, k, buf, par).wait()

        @pl.when(j + 3 < CH)
        def _():
            j3 = j + 3
            g3 = lax.div(j3, GRP)
            k3 = lax.rem(j3, GRP)

            @pl.when(k3 == 0)
            def _():
                for d_ in idx_group(g3, lax.rem(g3, 2)):
                    d_.wait()

            gather_desc(lax.rem(g3, 2), k3, lax.rem(j3, 4), lax.rem(j3, 3)).start()

        scatter_desc(pg, k, buf).start(add=True)
        return carry

    lax.fori_loop(0, CH, body, 0)
    jm = CH - 1
    scatter_desc((jm // GRP) % 2, jm % GRP, jm % 4).wait()
    plsc.subcore_barrier()
    pltpu.sync_copy(accum_sh.at[sl], out_hbm.at[c, sl])


def _sc_scatter(g, src3, dst3, zeros):
    return pl.kernel(
        _scatter_body,
        out_type=jax.ShapeDtypeStruct((NC, NP, D), jnp.float32),
        mesh=_mesh(),
        scratch_types=[
            pltpu.VMEM((2, GRP, CHUNK), jnp.int32),
            pltpu.VMEM((2, GRP, CHUNK), jnp.int32),
            pltpu.VMEM((4, CHUNK, D), jnp.float32),
            pltpu.VMEM_SHARED((NP, D), jnp.float32),
            pltpu.SemaphoreType.DMA((3,)),
            pltpu.SemaphoreType.DMA,
            pltpu.SemaphoreType.DMA,
        ],
    )(g, src3, dst3, zeros)


BM = 10240


def _dinv_of(cnt_ref):
    ones = jnp.ones((NW, 1), jnp.float32)
    deg = lax.dot_general(cnt_ref[...], ones, (((0,), (0,)), ((), ())),
                          preferred_element_type=jnp.float32) + 1.0
    return lax.rsqrt(deg)


def _stage_a_body(x_ref, w_ref, cnt_ref, o_ref):
    dinv = _dinv_of(cnt_ref)
    h = jnp.dot(x_ref[...], w_ref[...], preferred_element_type=jnp.float32)
    o_ref[...] = h * dinv


def _stage_b_body(p_ref, g_ref, cnt_ref, b_ref, w_ref, o_ref, dinv_ref):
    dinv = _dinv_of(cnt_ref)
    dinv_ref[...] = dinv
    s1 = p_ref[0] + p_ref[1] + g_ref[...]
    h1 = jnp.maximum(dinv * s1 + b_ref[...], 0.0)
    o_ref[...] = jnp.dot(h1, w_ref[...], preferred_element_type=jnp.float32) * dinv


def _stage_c_body(p_ref, g_ref, dinv_ref, b_ref, w_ref, bc_ref, o_ref):
    dinv = dinv_ref[...]
    agg = dinv * (p_ref[0] + p_ref[1] + g_ref[...]) + b_ref[...]
    o_ref[...] = (
        jnp.dot(agg, w_ref[...], preferred_element_type=jnp.float32) + bc_ref[...]
    )


def _row_spec(width):
    return pl.BlockSpec((BM, width), lambda i: (i, 0))


_CNT_SPEC = pl.BlockSpec((NW, BM), lambda i: (0, i))
_PART_SPEC = pl.BlockSpec((NC, BM, D), lambda i: (0, i, 0))
_W_SPEC = pl.BlockSpec((D, D), lambda i: (0, 0))
_B_SPEC = pl.BlockSpec((1, D), lambda i: (0, 0))


def _stage_a(x, W1, cnt):
    return pl.pallas_call(
        _stage_a_body,
        grid=(NP // BM,),
        in_specs=[_row_spec(D), _W_SPEC, _CNT_SPEC],
        out_specs=_row_spec(D),
        out_shape=jax.ShapeDtypeStruct((NP, D), jnp.float32),
    )(x, W1, cnt)


def _stage_b(p, g, cnt, b1, W2):
    return pl.pallas_call(
        _stage_b_body,
        grid=(NP // BM,),
        in_specs=[_PART_SPEC, _row_spec(D), _CNT_SPEC, _B_SPEC, _W_SPEC],
        out_specs=[_row_spec(D), pl.BlockSpec((BM, 1), lambda i: (i, 0))],
        out_shape=[jax.ShapeDtypeStruct((NP, D), jnp.float32),
                   jax.ShapeDtypeStruct((NP, 1), jnp.float32)],
    )(p, g, cnt, b1, W2)


def _stage_c(p, g, dinv, b2, Wc, bc):
    n_classes = Wc.shape[1]
    bm = N
    return pl.pallas_call(
        _stage_c_body,
        grid=(1,),
        in_specs=[
            pl.BlockSpec((NC, bm, D), lambda i: (0, i, 0)),
            pl.BlockSpec((bm, D), lambda i: (i, 0)),
            pl.BlockSpec((bm, 1), lambda i: (i, 0)),
            _B_SPEC,
            pl.BlockSpec((D, n_classes), lambda i: (0, 0)),
            pl.BlockSpec((1, n_classes), lambda i: (0, 0)),
        ],
        out_specs=pl.BlockSpec((bm, n_classes), lambda i: (i, 0)),
        out_shape=jax.ShapeDtypeStruct((N, n_classes), jnp.float32),
    )(p, g, dinv, b2, Wc, bc)


def kernel(x, edge_index, W1, b1, W2, b2, Wc, bc):
    n_classes = Wc.shape[1]
    src2 = edge_index[0].reshape(NW, E // NW)
    dst2 = edge_index[1].reshape(NW, E // NW)
    pad_w = EPW - E // NW
    pad_src = jnp.broadcast_to((jnp.arange(pad_w, dtype=jnp.int32) * 41) % N,
                               (NW, pad_w))
    pad_dst = jnp.broadcast_to(N + jnp.arange(pad_w, dtype=jnp.int32) % (NP - N),
                               (NW, pad_w))
    src3 = jnp.concatenate([src2, pad_src], axis=1).reshape(NW, CH, CHUNK)
    dst3 = jnp.concatenate([dst2, pad_dst], axis=1).reshape(NW, CH, CHUNK)

    zeros_d = jnp.zeros((NP, D), jnp.float32)
    b1r = b1.reshape(1, D)
    b2r = b2.reshape(1, D)
    bcr = bc.reshape(1, n_classes)

    cnt = _sc_count(dst3.reshape(NW, EPW))
    g1 = _stage_a(x, W1, cnt)
    p1 = _sc_scatter(g1, src3, dst3, zeros_d)
    g2, dinv = _stage_b(p1, g1, cnt, b1r, W2)
    p2 = _sc_scatter(g2, src3, dst3, zeros_d)
    return _stage_c(p2, g2, dinv, b2r, Wc, bcr)


# speedup vs baseline: 1.0134x; 1.0134x over previous
import jax
import jax.numpy as jnp
from jax import lax
from jax.experimental import pallas as pl
from jax.experimental.pallas import tpu as pltpu
from jax.experimental.pallas import tpu_sc as plsc

N = 10000
E = 320000
D = 128
NC = 2
NS = 16
NW = NC * NS
CHUNK = 80
NP = 10240
GRP = 8
CH = 128
NG = CH // GRP
EPW = CH * CHUNK
ROWS_PER_TILE = NP // NS


def _mesh():
    return plsc.VectorSubcoreMesh(core_axis_name="c", subcore_axis_name="s")


def _count_body(dst_hbm, out_hbm, idx_v, acc_v):
    c = lax.axis_index("c")
    s = lax.axis_index("s")
    wid = c * NS + s
    pltpu.sync_copy(dst_hbm.at[wid], idx_v)

    def zero(i, carry):
        acc_v[pl.ds(pl.multiple_of(i * 16, 16), 16)] = jnp.zeros((16,), jnp.float32)
        return carry

    lax.fori_loop(0, NP // 16, zero, 0)
    ones = jnp.ones((16,), jnp.float32)

    def body(i, carry):
        idx = idx_v[pl.ds(pl.multiple_of(i * 16, 16), 16)]
        plsc.addupdate_scatter(acc_v, [idx], ones)
        return carry

    lax.fori_loop(0, EPW // 16, body, 0)
    pltpu.sync_copy(acc_v, out_hbm.at[wid])


def _sc_count(dst2):
    return pl.kernel(
        _count_body,
        out_type=jax.ShapeDtypeStruct((NW, NP), jnp.float32),
        mesh=_mesh(),
        scratch_types=[
            pltpu.VMEM((EPW,), jnp.int32),
            pltpu.VMEM((NP,), jnp.float32),
        ],
        compiler_params=pltpu.CompilerParams(needs_layout_passes=False),
    )(dst2)


def _scatter_body(g_hbm, src_hbm, dst_hbm, zeros_hbm, out_hbm,
                  isrc_v, idst_v, rows_v, accum_sh, gsems, ssem, isem):
    c = lax.axis_index("c")
    s = lax.axis_index("s")
    wid = c * NS + s
    sl = pl.ds(s * ROWS_PER_TILE, ROWS_PER_TILE)

    def idx_group(grp, pbuf):
        return (
            pltpu.make_async_copy(
                src_hbm.at[wid, pl.ds(grp * GRP, GRP)], isrc_v.at[pbuf], isem),
            pltpu.make_async_copy(
                dst_hbm.at[wid, pl.ds(grp * GRP, GRP)], idst_v.at[pbuf], isem),
        )

    def gather_desc(pg, k, buf, par):
        return pltpu.make_async_copy(
            g_hbm.at[isrc_v.at[pg, k]], rows_v.at[buf], gsems.at[par])

    def scatter_desc(pg, k, buf):
        return pltpu.make_async_copy(
            rows_v.at[buf], accum_sh.at[idst_v.at[pg, k]], ssem)

    for d_ in idx_group(0, 0):
        d_.start()
    pltpu.sync_copy(zeros_hbm.at[sl], accum_sh.at[sl])
    for d_ in idx_group(0, 0):
        d_.wait()
    for d_ in idx_group(1, 1):
        d_.start()
    gather_desc(0, 0, 0, 0).start()
    gather_desc(0, 1, 1, 1).start()
    gather_desc(0, 2, 2, 2).start()
    plsc.subcore_barrier()

    def body(j, carry):
        grp = lax.div(j, GRP)
        k = lax.rem(j, GRP)
        pg = lax.rem(grp, 2)
        buf = lax.rem(j, 4)
        par = lax.rem(j, 3)

        @pl.when(jnp.logical_and(jnp.logical_and(k == 0, j > 0), grp + 1 < NG))
        def _():
            for d_ in idx_group(grp + 1, lax.rem(grp + 1, 2)):
                d_.start()

        @pl.when(j > 0)
        def _():
            jm = j - 1
            scatter_desc(lax.rem(lax.div(jm, GRP), 2), lax.rem(jm, GRP),
                         lax.rem(jm, 4)).wait()

        gather_desc(pg, k, buf, par).wait()

        @pl.when(j + 3 < CH)
        def _():
            j3 = j + 3
            g3 = lax.div(j3, GRP)
            k3 = lax.rem(j3, GRP)

            @pl.when(k3 == 0)
            def _():
                for d_ in idx_group(g3, lax.rem(g3, 2)):
                    d_.wait()

            gather_desc(lax.rem(g3, 2), k3, lax.rem(j3, 4), lax.rem(j3, 3)).start()

        scatter_desc(pg, k, buf).start(add=True)
        return carry

    lax.fori_loop(0, CH, body, 0)
    jm = CH - 1
    scatter_desc((jm // GRP) % 2, jm % GRP, jm % 4).wait()
    plsc.subcore_barrier()
    pltpu.sync_copy(accum_sh.at[sl], out_hbm.at[c, sl])


def _sc_scatter(g, src3, dst3, zeros):
    return pl.kernel(
        _scatter_body,
        out_type=jax.ShapeDtypeStruct((NC, NP, D), jnp.float32),
        mesh=_mesh(),
        scratch_types=[
            pltpu.VMEM((2, GRP, CHUNK), jnp.int32),
            pltpu.VMEM((2, GRP, CHUNK), jnp.int32),
            pltpu.VMEM((4, CHUNK, D), jnp.float32),
            pltpu.VMEM_SHARED((NP, D), jnp.float32),
            pltpu.SemaphoreType.DMA((3,)),
            pltpu.SemaphoreType.DMA,
            pltpu.SemaphoreType.DMA,
        ],
    )(g, src3, dst3, zeros)


BM = 5120


def _dinv_of(cnt_ref):
    ones = jnp.ones((NW, 1), jnp.float32)
    deg = lax.dot_general(cnt_ref[...], ones, (((0,), (0,)), ((), ())),
                          preferred_element_type=jnp.float32) + 1.0
    return lax.rsqrt(deg)


def _stage_a_body(x_ref, w_ref, cnt_ref, o_ref):
    dinv = _dinv_of(cnt_ref)
    h = jnp.dot(x_ref[...], w_ref[...], preferred_element_type=jnp.float32)
    o_ref[...] = h * dinv


def _stage_b_body(p_ref, g_ref, cnt_ref, b_ref, w_ref, o_ref, dinv_ref):
    dinv = _dinv_of(cnt_ref)
    dinv_ref[...] = dinv
    s1 = p_ref[0] + p_ref[1] + g_ref[...]
    h1 = jnp.maximum(dinv * s1 + b_ref[...], 0.0)
    o_ref[...] = jnp.dot(h1, w_ref[...], preferred_element_type=jnp.float32) * dinv


def _stage_c_body(p_ref, g_ref, dinv_ref, b_ref, w_ref, bc_ref, o_ref):
    dinv = dinv_ref[...]
    agg = dinv * (p_ref[0] + p_ref[1] + g_ref[...]) + b_ref[...]
    o_ref[...] = (
        jnp.dot(agg, w_ref[...], preferred_element_type=jnp.float32) + bc_ref[...]
    )


def _row_spec(width):
    return pl.BlockSpec((BM, width), lambda i: (i, 0))


_CNT_SPEC = pl.BlockSpec((NW, BM), lambda i: (0, i))
_PART_SPEC = pl.BlockSpec((NC, BM, D), lambda i: (0, i, 0))
_W_SPEC = pl.BlockSpec((D, D), lambda i: (0, 0))
_B_SPEC = pl.BlockSpec((1, D), lambda i: (0, 0))


def _stage_a(x, W1, cnt):
    return pl.pallas_call(
        _stage_a_body,
        grid=(NP // BM,),
        in_specs=[_row_spec(D), _W_SPEC, _CNT_SPEC],
        out_specs=_row_spec(D),
        out_shape=jax.ShapeDtypeStruct((NP, D), jnp.float32),
    )(x, W1, cnt)


def _stage_b(p, g, cnt, b1, W2):
    return pl.pallas_call(
        _stage_b_body,
        grid=(NP // BM,),
        in_specs=[_PART_SPEC, _row_spec(D), _CNT_SPEC, _B_SPEC, _W_SPEC],
        out_specs=[_row_spec(D), pl.BlockSpec((BM, 1), lambda i: (i, 0))],
        out_shape=[jax.ShapeDtypeStruct((NP, D), jnp.float32),
                   jax.ShapeDtypeStruct((NP, 1), jnp.float32)],
    )(p, g, cnt, b1, W2)


def _stage_c(p, g, dinv, b2, Wc, bc):
    n_classes = Wc.shape[1]
    bm = N // 2
    return pl.pallas_call(
        _stage_c_body,
        grid=(2,),
        in_specs=[
            pl.BlockSpec((NC, bm, D), lambda i: (0, i, 0)),
            pl.BlockSpec((bm, D), lambda i: (i, 0)),
            pl.BlockSpec((bm, 1), lambda i: (i, 0)),
            _B_SPEC,
            pl.BlockSpec((D, n_classes), lambda i: (0, 0)),
            pl.BlockSpec((1, n_classes), lambda i: (0, 0)),
        ],
        out_specs=pl.BlockSpec((bm, n_classes), lambda i: (i, 0)),
        out_shape=jax.ShapeDtypeStruct((N, n_classes), jnp.float32),
    )(p, g, dinv, b2, Wc, bc)


def kernel(x, edge_index, W1, b1, W2, b2, Wc, bc):
    n_classes = Wc.shape[1]
    src2 = edge_index[0].reshape(NW, E // NW)
    dst2 = edge_index[1].reshape(NW, E // NW)
    pad_w = EPW - E // NW
    pad_src = jnp.broadcast_to((jnp.arange(pad_w, dtype=jnp.int32) * 41) % N,
                               (NW, pad_w))
    pad_dst = jnp.broadcast_to(N + jnp.arange(pad_w, dtype=jnp.int32) % (NP - N),
                               (NW, pad_w))
    src3 = jnp.concatenate([src2, pad_src], axis=1).reshape(NW, CH, CHUNK)
    dst3 = jnp.concatenate([dst2, pad_dst], axis=1).reshape(NW, CH, CHUNK)

    zeros_d = jnp.zeros((NP, D), jnp.float32)
    b1r = b1.reshape(1, D)
    b2r = b2.reshape(1, D)
    bcr = bc.reshape(1, n_classes)

    cnt = _sc_count(dst3.reshape(NW, EPW))
    g1 = _stage_a(x, W1, cnt)
    p1 = _sc_scatter(g1, src3, dst3, zeros_d)
    g2, dinv = _stage_b(p1, g1, cnt, b1r, W2)
    p2 = _sc_scatter(g2, src3, dst3, zeros_d)
    return _stage_c(p2, g2, dinv, b2r, Wc, bcr)
